# baseline (speedup 1.0000x reference)
import functools

import jax
import jax.numpy as jnp
from jax import lax
from jax.experimental import pallas as pl
from jax.experimental.pallas import tpu as pltpu
from jax.experimental.pallas import tpu_sc as plsc

_info = plsc.get_sparse_core_info()
_NC = _info.num_cores
_NS = _info.num_subcores
_NW = _NC * _NS


def _make_gather(N, V, DP, D, chunk):
    n_per_w = N // _NW
    n_chunks = n_per_w // chunk
    n_chunks_total = N // chunk
    mesh = plsc.VectorSubcoreMesh(core_axis_name="c", subcore_axis_name="s")

    @functools.partial(
        pl.kernel,
        mesh=mesh,
        out_type=jax.ShapeDtypeStruct((n_chunks_total, chunk, D), jnp.float32),
        scratch_types=[
            pltpu.VMEM((n_chunks, chunk), jnp.int32),
            pltpu.VMEM((chunk, DP), jnp.float32),
            pltpu.VMEM((chunk, DP), jnp.float32),
            pltpu.SemaphoreType.DMA,
            pltpu.SemaphoreType.DMA,
        ],
        compiler_params=pltpu.CompilerParams(use_tc_tiling_on_sc=False),
    )
    def gather_kernel(idx_hbm, table_hbm, out_hbm, idx_all, buf0, buf1,
                      sem0, sem1):
        wid = lax.axis_index("s") * _NC + lax.axis_index("c")
        g0 = wid * n_chunks
        pltpu.sync_copy(idx_hbm.at[pl.ds(g0, n_chunks)], idx_all)

        bufs = (buf0, buf1)
        sems = (sem0, sem1)
        handles = [None] * n_chunks
        handles[0] = pltpu.async_copy(
            table_hbm.at[idx_all.at[0]], bufs[0], sems[0])
        for j in range(n_chunks):
            if j + 1 < n_chunks:
                handles[j + 1] = pltpu.async_copy(
                    table_hbm.at[idx_all.at[j + 1]],
                    bufs[(j + 1) % 2], sems[(j + 1) % 2])
            handles[j].wait()
            pltpu.sync_copy(bufs[j % 2].at[:, pl.ds(0, D)], out_hbm.at[g0 + j])

    return gather_kernel


def kernel(indices, table):
    B, F = indices.shape
    V, D = table.shape
    N = B * F
    chunk = 416
    DP = 128
    flat_idx = indices.reshape(N // chunk, chunk).astype(jnp.int32)
    tpad = jnp.pad(table, ((0, 0), (0, DP - D)))
    out = _make_gather(N, V, DP, D, chunk)(flat_idx, tpad)
    return out.reshape(B, F, D)


# speedup vs baseline: 1.0622x; 1.0622x over previous
import functools

import jax
import jax.numpy as jnp
from jax import lax
from jax.experimental import pallas as pl
from jax.experimental.pallas import tpu as pltpu
from jax.experimental.pallas import tpu_sc as plsc

_info = plsc.get_sparse_core_info()
_NC = _info.num_cores
_NS = _info.num_subcores
_NW = _NC * _NS


def _make_gather(B, F, V, D):
    N = B * F
    chunk = 32 * F
    n_chunks = N // (chunk * _NW)
    mesh = plsc.VectorSubcoreMesh(core_axis_name="c", subcore_axis_name="s")

    @functools.partial(
        pl.kernel,
        mesh=mesh,
        out_type=jax.ShapeDtypeStruct((N, D), jnp.float32),
        scratch_types=[
            pltpu.VMEM((n_chunks, chunk), jnp.int32),
            pltpu.VMEM((chunk, D), jnp.float32),
            pltpu.VMEM((chunk, D), jnp.float32),
            pltpu.SemaphoreType.DMA,
            pltpu.SemaphoreType.DMA,
        ],
        compiler_params=pltpu.CompilerParams(use_tc_tiling_on_sc=False),
    )
    def gather_kernel(idx_hbm, table_hbm, out_hbm, idx_all, buf0, buf1,
                      sem0, sem1):
        wid = lax.axis_index("s") * _NC + lax.axis_index("c")
        pltpu.sync_copy(idx_hbm.at[pl.ds(wid * n_chunks, n_chunks)], idx_all)

        bufs = (buf0, buf1)
        sems = (sem0, sem1)

        handles = [None] * n_chunks
        handles[0] = pltpu.async_copy(
            table_hbm.at[idx_all.at[0]], bufs[0], sems[0])
        for j in range(n_chunks):
            if j + 1 < n_chunks:
                handles[j + 1] = pltpu.async_copy(
                    table_hbm.at[idx_all.at[j + 1]],
                    bufs[(j + 1) % 2], sems[(j + 1) % 2])
            handles[j].wait()
            base = (wid * n_chunks + j) * chunk
            pltpu.sync_copy(bufs[j % 2], out_hbm.at[pl.ds(base, chunk)])

    return gather_kernel


def kernel(indices, table):
    B, F = indices.shape
    V, D = table.shape
    N = B * F
    chunk = 32 * F
    flat_idx = indices.reshape(N // chunk, chunk).astype(jnp.int32)
    out = _make_gather(B, F, V, D)(flat_idx, table)
    return out.reshape(B, F, D)


# speedup vs baseline: 1.0626x; 1.0004x over previous
import functools

import jax
import jax.numpy as jnp
from jax import lax
from jax.experimental import pallas as pl
from jax.experimental.pallas import tpu as pltpu
from jax.experimental.pallas import tpu_sc as plsc

_info = plsc.get_sparse_core_info()
_NC = _info.num_cores
_NS = _info.num_subcores
_NW = _NC * _NS


def _make_gather(B, F, V, D):
    BLQ = 32
    chunk = BLQ * F
    b_per_w = B // _NW
    n_chunks = b_per_w // BLQ
    mesh = plsc.VectorSubcoreMesh(core_axis_name="c", subcore_axis_name="s")

    @functools.partial(
        pl.kernel,
        mesh=mesh,
        out_type=jax.ShapeDtypeStruct((B * F, D), jnp.float32),
        scratch_types=(
            [pltpu.VMEM((n_chunks * chunk,), jnp.int32)]
            + [pltpu.VMEM((chunk, D), jnp.float32)] * 4
            + [pltpu.SemaphoreType.DMA] * 8
        ),
        compiler_params=pltpu.CompilerParams(use_tc_tiling_on_sc=False),
    )
    def gather_kernel(idx_hbm, table_hbm, out_hbm, idx_all,
                      buf0, buf1, buf2, buf3,
                      gs0, gs1, gs2, gs3, ws0, ws1, ws2, ws3):
        wid = lax.axis_index("s") * _NC + lax.axis_index("c")
        pltpu.sync_copy(
            idx_hbm.at[pl.ds(wid * n_chunks * chunk, n_chunks * chunk)],
            idx_all)

        bufs = (buf0, buf1, buf2, buf3)
        gsems = (gs0, gs1, gs2, gs3)
        wsems = (ws0, ws1, ws2, ws3)

        def gather(j):
            return pltpu.async_copy(
                table_hbm.at[idx_all.at[pl.ds(j * chunk, chunk)]],
                bufs[j % 4], gsems[j % 4])

        gh = [None] * n_chunks
        wh = [None] * n_chunks
        for j in range(min(4, n_chunks)):
            gh[j] = gather(j)
        for j in range(n_chunks):
            gh[j].wait()
            base = (wid * n_chunks + j) * chunk
            wh[j] = pltpu.async_copy(
                bufs[j % 4], out_hbm.at[pl.ds(base, chunk)], wsems[j % 4])
            if j + 4 < n_chunks:
                wh[j].wait()
                gh[j + 4] = gather(j + 4)
        for j in range(max(0, n_chunks - 4), n_chunks):
            if wh[j] is not None:
                wh[j].wait()

    return gather_kernel


def kernel(indices, table):
    B, F = indices.shape
    V, D = table.shape
    N = B * F
    chunk = 32 * F
    flat_idx = indices.reshape(N).astype(jnp.int32)
    out = _make_gather(B, F, V, D)(flat_idx, table)
    return out.reshape(B, F, D)
